# Initial kernel scaffold; baseline (speedup 1.0000x reference)
import jax
import jax.numpy as jnp
from jax.experimental import pallas as pl


def kernel(x, edge_index, batch, observable_features, noise_factor, noisy_exp, Wl1, Wr1, att1, b1, Wl2, Wr2, att2, b2, obs_W1, obs_b1, obs_W2, obs_b2, noise_W, noise_b, fus_W1, fus_b1, fus_W2, fus_b2):
    raise NotImplementedError("write your pallas kernel here")


# speedup vs baseline: 6.4694x; 6.4694x over previous
import functools

import jax
import jax.numpy as jnp
from jax import lax
from jax.experimental import pallas as pl
from jax.experimental.pallas import tpu as pltpu
from jax.experimental.pallas import tpu_sc as plsc

def _perm16(v, idx):
    return lax.gather(
        v, idx[:, None],
        lax.GatherDimensionNumbers(offset_dims=(), collapsed_slice_dims=(0,),
                                   start_index_map=(0,)),
        (1,), mode=lax.GatherScatterMode.PROMISE_IN_BOUNDS)


_N = 10000
_NPAD = 10240
_E = 320000
_ETOT = _E + _N
_K = 128
_NTILES = 32
_CH = 81
_EPAD = _NTILES * _CH * _K
_ROWS_PER_TILE = _NPAD // 16
_B = 64


def _proj_body(x_ref, wl_ref, wr_ref, xl_ref, xr_ref):
    xb = x_ref[...]
    xl_ref[...] = jnp.dot(xb, wl_ref[...], preferred_element_type=jnp.float32)
    xr_ref[...] = jnp.dot(xb, wr_ref[...], preferred_element_type=jnp.float32)


def _project(xp, wl, wr):
    din, dout = wl.shape
    return pl.pallas_call(
        _proj_body,
        grid=(10,),
        in_specs=[
            pl.BlockSpec((1024, din), lambda i: (i, 0)),
            pl.BlockSpec((din, dout), lambda i: (0, 0)),
            pl.BlockSpec((din, dout), lambda i: (0, 0)),
        ],
        out_specs=[
            pl.BlockSpec((1024, dout), lambda i: (i, 0)),
            pl.BlockSpec((1024, dout), lambda i: (i, 0)),
        ],
        out_shape=[
            jax.ShapeDtypeStruct((_NPAD, dout), jnp.float32),
            jax.ShapeDtypeStruct((_NPAD, dout), jnp.float32),
        ],
    )(xp, wl, wr)


def _make_edge_kernel(d, nodesplit):
    dw = d + 16
    nj = d // 16
    if nodesplit:
        nl = _NPAD // 2
        nrows = 5248
        ch = _EPAD // (16 * _K)
    else:
        nl = 0
        nrows = _NPAD
        ch = _EPAD // (32 * _K)
    rows_pt = nrows // 16
    mesh = plsc.VectorSubcoreMesh(core_axis_name="c", subcore_axis_name="s")

    @functools.partial(
        pl.kernel,
        mesh=mesh,
        compiler_params=pltpu.CompilerParams(use_tc_tiling_on_sc=False),
        out_type=jax.ShapeDtypeStruct((2 * nrows, dw), jnp.float32),
        scratch_types=[
            pltpu.VMEM((_K,), jnp.int32),
            pltpu.VMEM((_K,), jnp.int32),
            pltpu.VMEM((_K,), jnp.int32),
            pltpu.VMEM((_K, d), jnp.float32),
            pltpu.VMEM((_K, d), jnp.float32),
            pltpu.VMEM((_K, dw), jnp.float32),
            pltpu.VMEM((d,), jnp.float32),
            pltpu.VMEM_SHARED((nrows, dw), jnp.float32),
            pltpu.SemaphoreType.DMA,
        ],
    )
    def k(xl_hbm, xr_hbm, src_hbm, dst_hbm, att_hbm, out_hbm,
          sidx, didx, lidx, a_v, b_v, c_v, att_v, acc, sem):
        cid = lax.axis_index("c")
        sid = lax.axis_index("s")
        zero16 = jnp.zeros((16,), jnp.float32)

        def zrow(r, _):
            for j in range(dw // 16):
                c_v[r, pl.ds(16 * j, 16)] = zero16
            return 0
        lax.fori_loop(0, _K, zrow, 0)

        off = 0
        rem = rows_pt
        while rem > 0:
            sz = min(_K, rem)
            pltpu.sync_copy(c_v.at[pl.ds(0, sz)],
                            acc.at[pl.ds(sid * rows_pt + off, sz)])
            off += sz
            rem -= sz
        plsc.subcore_barrier()

        pltpu.sync_copy(att_hbm, att_v)
        att_regs = [att_v[pl.ds(16 * j, 16)] for j in range(nj)]
        lanes = lax.iota(jnp.int32, 16)
        mask0 = lanes == 0
        perms = [lanes ^ sh for sh in (1, 2, 4, 8)]

        if nodesplit:
            base_t = sid * (ch * _K)
        else:
            base_t = (sid * 2 + cid) * (ch * _K)

        def chunk_body(c, _):
            base = base_t + c * _K
            pltpu.sync_copy(src_hbm.at[pl.ds(base, _K)], sidx)
            pltpu.sync_copy(dst_hbm.at[pl.ds(base, _K)], didx)
            if nodesplit:
                lo = cid * nl
                for i in range(_K // 16):
                    dv = didx[pl.ds(16 * i, 16)]
                    lv = dv - lo
                    inb = (lv >= 0) & (lv < nl)
                    lidx[pl.ds(16 * i, 16)] = jnp.where(inb, lv, nl)
                scatter_idx = lidx
            else:
                scatter_idx = didx
            pltpu.async_copy(xl_hbm.at[sidx], a_v, sem).wait()
            pltpu.async_copy(xr_hbm.at[didx], b_v, sem).wait()

            def edge_body(e, _):
                avs = []
                p = None
                for j in range(nj):
                    av = a_v[e, pl.ds(16 * j, 16)]
                    bv = b_v[e, pl.ds(16 * j, 16)]
                    t = av + bv
                    t = jnp.maximum(t, 0.2 * t)
                    q = t * att_regs[j]
                    p = q if p is None else p + q
                    avs.append(av)
                for pidx in perms:
                    p = p + _perm16(p, pidx)
                wv = jnp.exp(p)
                for j in range(nj):
                    c_v[e, pl.ds(16 * j, 16)] = avs[j] * wv
                c_v[e, pl.ds(d, 16)] = jnp.where(mask0, wv, zero16)
                return 0
            lax.fori_loop(0, _K, edge_body, 0)
            pltpu.sync_copy(c_v, acc.at[scatter_idx], add=True)
            return 0
        lax.fori_loop(0, ch, chunk_body, 0)
        plsc.subcore_barrier()

        pltpu.sync_copy(
            acc.at[pl.ds(sid * rows_pt, rows_pt)],
            out_hbm.at[pl.ds(cid * nrows + sid * rows_pt, rows_pt)])

    return k


_edge128 = _make_edge_kernel(128, nodesplit=True)
_edge32 = _make_edge_kernel(32, nodesplit=False)


def _mid_body(p_ref, b_ref, wl_ref, wr_ref, xl_ref, xr_ref):
    s = p_ref[...]
    den = jnp.maximum(s[:, 128:129], 1e-30)
    h = s[:, :128] / den + b_ref[...]
    h = jnp.where(h > 0, h, jnp.exp(jnp.minimum(h, 0.0)) - 1.0)
    xl_ref[...] = jnp.dot(h, wl_ref[...], preferred_element_type=jnp.float32)
    xr_ref[...] = jnp.dot(h, wr_ref[...], preferred_element_type=jnp.float32)


def _mid(p, b1, wl2, wr2):
    return pl.pallas_call(
        _mid_body,
        grid=(10,),
        in_specs=[
            pl.BlockSpec((1024, 144), lambda i: (i, 0)),
            pl.BlockSpec((1, 128), lambda i: (0, 0)),
            pl.BlockSpec((128, 32), lambda i: (0, 0)),
            pl.BlockSpec((128, 32), lambda i: (0, 0)),
        ],
        out_specs=[
            pl.BlockSpec((1024, 32), lambda i: (i, 0)),
            pl.BlockSpec((1024, 32), lambda i: (i, 0)),
        ],
        out_shape=[
            jax.ShapeDtypeStruct((_NPAD, 32), jnp.float32),
            jax.ShapeDtypeStruct((_NPAD, 32), jnp.float32),
        ],
    )(p, b1, wl2, wr2)


def _head_body(q0_ref, q1_ref, batch_ref, obs_ref, noise_ref, noisy_ref,
               b2_ref, ow1_ref, ob1_ref, ow2_ref, ob2_ref,
               nw_ref, nb_ref, fw1_ref, fb1_ref, fw2_ref, fb2_ref, out_ref):
    s = q0_ref[...] + q1_ref[...]
    den = jnp.maximum(s[:, 32:33], 1e-30)
    h = s[:, :32] / den + b2_ref[...]
    h = jnp.where(h > 0, h, jnp.exp(jnp.minimum(h, 0.0)) - 1.0)

    bt = batch_ref[...]
    gids = lax.broadcasted_iota(jnp.int32, (_NPAD, _B), 1)
    onehot = (bt == gids).astype(jnp.float32)
    sums = lax.dot_general(onehot, h, (((0,), (0,)), ((), ())),
                           preferred_element_type=jnp.float32)
    cnt = jnp.sum(onehot, axis=0)[:, None]
    circ = sums / jnp.maximum(cnt, 1.0)

    obs = obs_ref[...]
    t = jnp.dot(obs, ow1_ref[...], preferred_element_type=jnp.float32)
    t = jnp.maximum(t + ob1_ref[...], 0.0)
    obs_e = jnp.dot(t, ow2_ref[...],
                    preferred_element_type=jnp.float32) + ob2_ref[...]

    noise_e = jnp.dot(noise_ref[...], nw_ref[...],
                      preferred_element_type=jnp.float32) + nb_ref[...]

    noisy = noisy_ref[...]
    comb = jnp.concatenate([circ, obs_e, noise_e, noisy], axis=1)
    u = jnp.dot(comb, fw1_ref[...], preferred_element_type=jnp.float32)
    u = jnp.maximum(u + fb1_ref[...], 0.0)
    corr = jnp.dot(u, fw2_ref[...],
                   preferred_element_type=jnp.float32) + fb2_ref[...]
    out_ref[...] = noisy + corr


def _head(q0, q1, batch2d, obs, noise, noisy, b2,
          ow1, ob1, ow2, ob2, nw, nb, fw1, fb1, fw2, fb2):
    return pl.pallas_call(
        _head_body,
        out_shape=jax.ShapeDtypeStruct((_B, 1), jnp.float32),
    )(q0, q1, batch2d, obs, noise, noisy, b2,
      ow1, ob1, ow2, ob2, nw, nb, fw1, fb1, fw2, fb2)


def kernel(x, edge_index, batch, observable_features, noise_factor, noisy_exp,
           Wl1, Wr1, att1, b1, Wl2, Wr2, att2, b2,
           obs_W1, obs_b1, obs_W2, obs_b2, noise_W, noise_b,
           fus_W1, fus_b1, fus_W2, fus_b2):
    xp = jnp.zeros((_NPAD, 128), jnp.float32).at[:_N].set(x)
    loop = jnp.arange(_N, dtype=jnp.int32)
    pad = jnp.full((_EPAD - _ETOT,), _N, jnp.int32)
    src = jnp.concatenate([edge_index[0].astype(jnp.int32), loop, pad])
    dst = jnp.concatenate([edge_index[1].astype(jnp.int32), loop, pad])

    xl1, xr1 = _project(xp, Wl1, Wr1)
    part1 = _edge128(xl1, xr1, src, dst, att1)
    p1 = jnp.concatenate([part1[:_NPAD // 2], part1[5248:5248 + _NPAD // 2]])
    xl2, xr2 = _mid(p1, b1.reshape(1, 128), Wl2, Wr2)
    part2 = _edge32(xl2, xr2, src, dst, att2)

    batch2d = jnp.concatenate(
        [batch.astype(jnp.int32), jnp.full((_NPAD - _N,), _B, jnp.int32)]
    ).reshape(_NPAD, 1)
    return _head(part2[:_NPAD], part2[_NPAD:], batch2d,
                 observable_features[:, 0, :], noise_factor, noisy_exp,
                 b2.reshape(1, 32),
                 obs_W1, obs_b1.reshape(1, 32), obs_W2, obs_b2.reshape(1, 8),
                 noise_W, noise_b.reshape(1, 4),
                 fus_W1, fus_b1.reshape(1, 256), fus_W2, fus_b2.reshape(1, 1))
